# Initial kernel scaffold; baseline (speedup 1.0000x reference)
import jax
import jax.numpy as jnp
from jax.experimental import pallas as pl


def kernel(x, edge_attr_down, edge_attr_up, edge_index_down, edge_index_up):
    raise NotImplementedError("write your pallas kernel here")


# speedup vs baseline: 13.2198x; 13.2198x over previous
import functools

import jax
import jax.numpy as jnp
from jax import lax
from jax.experimental import pallas as pl
from jax.experimental.pallas import tpu as pltpu, tpu_sc as plsc

_ND = 50000
_NH = 10000
_F = 256
_FC = 32
_NC = _F // _FC
_K = 128
_NSUB = 16
_NCORE = 2
_ZB = 80


def _sc_body(nchunks, ncombo, x_hbm, edd_hbm, edu_hbm, out_hbm,
             hid_acc, out_acc, echunk, idx_g, rows_v, zeros_v, sem):
    core = lax.axis_index("c")
    sid = lax.axis_index("s")

    zvec = jnp.zeros((16,), jnp.float32)

    def _zero_fill(i, c):
        zeros_v[i, pl.ds(0, 16)] = zvec
        zeros_v[i, pl.ds(16, 16)] = zvec
        return c

    lax.fori_loop(0, _ZB, _zero_fill, 0)

    hid_blocks = _NH // _ZB
    out_blocks = _ND // _ZB

    def _blocked(nblocks, fn):
        def step(z, c):
            b = sid + z * _NSUB

            @pl.when(b < nblocks)
            def _():
                fn(pl.multiple_of(b * _ZB, 8))

            return c

        lax.fori_loop(0, -(-nblocks // _NSUB), step, 0)

    def _edge_pass(table, ed_hbm, acc, goff):
        def chunk(j, c):
            pltpu.sync_copy(ed_hbm.at[sid, j], echunk)
            if goff is None:
                gather_idx = echunk.at[0]
            else:
                for i in range(_K // 16):
                    idx_g[pl.ds(i * 16, 16)] = echunk[0, pl.ds(i * 16, 16)] + goff
                gather_idx = idx_g
            pltpu.async_copy(table.at[gather_idx], rows_v, sem).wait()

            def scale(g, c2):
                avec = plsc.bitcast(echunk[2, pl.ds(g * 16, 16)], jnp.float32)
                e0 = g * 16
                for ei in range(16):
                    a = avec[ei]
                    rows_v[e0 + ei, pl.ds(0, 16)] = rows_v[e0 + ei, pl.ds(0, 16)] * a
                    rows_v[e0 + ei, pl.ds(16, 16)] = rows_v[e0 + ei, pl.ds(16, 16)] * a
                return c2

            lax.fori_loop(0, _K // 16, scale, 0)
            pltpu.sync_copy(rows_v, acc.at[echunk.at[1]], add=True)
            return c

        lax.fori_loop(0, nchunks, chunk, 0)

    for mi in range(ncombo):
        m = core * ncombo + mi

        _blocked(hid_blocks,
                 lambda off: pltpu.sync_copy(zeros_v, hid_acc.at[pl.ds(off, _ZB)]))
        _blocked(out_blocks,
                 lambda off: pltpu.sync_copy(zeros_v, out_acc.at[pl.ds(off, _ZB)]))
        plsc.subcore_barrier()

        _edge_pass(x_hbm, edd_hbm, hid_acc, m * _ND)
        plsc.subcore_barrier()

        _edge_pass(hid_acc, edu_hbm, out_acc, None)
        plsc.subcore_barrier()

        _blocked(out_blocks,
                 lambda off: pltpu.sync_copy(
                     out_acc.at[pl.ds(off, _ZB)],
                     out_hbm.at[pl.ds(m * _ND + off, _ZB)]))
        plsc.subcore_barrier()


def kernel(x, edge_attr_down, edge_attr_up, edge_index_down, edge_index_up):
    batch, _, ens, nd, f = x.shape
    be = batch * ens
    ncombo = _NC * be // _NCORE
    e = edge_index_down.shape[1]
    nchunks = -(-e // (_NSUB * _K))
    epad = _NSUB * _K * nchunks

    def prep(edge_index, edge_attr):
        src = jnp.pad(edge_index[0].astype(jnp.int32), (0, epad - e))
        dst = jnp.pad(edge_index[1].astype(jnp.int32), (0, epad - e))
        attr = jax.lax.bitcast_convert_type(
            jnp.pad(edge_attr, (0, epad - e)), jnp.int32)
        ed = jnp.stack([src, dst, attr], axis=1)
        ed = ed.reshape(_NSUB, nchunks, _K, 3)
        return ed.transpose(0, 1, 3, 2)

    edd = prep(edge_index_down, edge_attr_down)
    edu = prep(edge_index_up, edge_attr_up)

    flat = x[:, -1].reshape(be, nd, _NC, _FC)
    x_t = flat.transpose(2, 0, 1, 3).reshape(_NC * be * nd, _FC)

    mesh = plsc.VectorSubcoreMesh(core_axis_name="c", subcore_axis_name="s")
    body = functools.partial(_sc_body, nchunks, ncombo)
    out = pl.kernel(
        body,
        out_type=jax.ShapeDtypeStruct((_NC * be * nd, _FC), jnp.float32),
        mesh=mesh,
        compiler_params=pltpu.CompilerParams(use_tc_tiling_on_sc=False,
                                             needs_layout_passes=False),
        scratch_types=[
            pltpu.VMEM_SHARED((_NH, _FC), jnp.float32),
            pltpu.VMEM_SHARED((_ND, _FC), jnp.float32),
            pltpu.VMEM((3, _K), jnp.int32),
            pltpu.VMEM((_K,), jnp.int32),
            pltpu.VMEM((_K, _FC), jnp.float32),
            pltpu.VMEM((_ZB, _FC), jnp.float32),
            pltpu.SemaphoreType.DMA,
        ],
    )(x_t, edd, edu)

    out = out.reshape(_NC, be, nd, _FC).transpose(1, 2, 0, 3)
    return out.reshape(batch, ens, nd, _F)


# speedup vs baseline: 18.0504x; 1.3654x over previous
import functools

import jax
import jax.numpy as jnp
from jax import lax
from jax.experimental import pallas as pl
from jax.experimental.pallas import tpu as pltpu, tpu_sc as plsc

_ND = 50000
_NH = 10000
_F = 256
_FC = 32
_NC = _F // _FC
_K = 128
_NSUB = 16
_NCORE = 2
_ZB = 40


def _sc_body(nchunks, ncombo, x_hbm, edd_hbm, edu_hbm, out_hbm,
             hid_acc, out_acc, ech, idxg, rows, zeros_v,
             sem_e0, sem_e1, sem_g0, sem_g1, sem_z, sem_f):
    core = lax.axis_index("c")
    sid = lax.axis_index("s")
    sem_e = (sem_e0, sem_e1)
    sem_g = (sem_g0, sem_g1)

    zvec = jnp.zeros((16,), jnp.float32)

    def _zero_fill(i, c):
        zeros_v[i, pl.ds(0, 16)] = zvec
        zeros_v[i, pl.ds(16, 16)] = zvec
        return c

    lax.fori_loop(0, _ZB, _zero_fill, 0)

    hid_blocks = _NH // _ZB
    out_blocks = _ND // _ZB

    def _burst(nblocks, fire):
        def step(z, c):
            b = sid + z * _NSUB

            @pl.when(b < nblocks)
            def _():
                fire(pl.multiple_of(b * _ZB, 8))

            return c

        lax.fori_loop(0, -(-nblocks // _NSUB), step, 0)

    def _drain(nblocks, mk):
        cnt = (nblocks - sid + _NSUB - 1) // _NSUB

        def step(z, c):
            mk().wait()
            return c

        lax.fori_loop(0, cnt, step, 0)

    def _zero_hid():
        _burst(hid_blocks, lambda off: pltpu.async_copy(
            zeros_v, hid_acc.at[pl.ds(off, _ZB)], sem_z))

    def _zero_out():
        _burst(out_blocks, lambda off: pltpu.async_copy(
            zeros_v, out_acc.at[pl.ds(off, _ZB)], sem_z))

    def _drain_zero(nblocks, acc):
        _drain(nblocks, lambda: pltpu.make_async_copy(
            zeros_v, acc.at[pl.ds(0, _ZB)], sem_z))

    def _edge_pass(table, ed_hbm, acc, goff):
        def _gidx(p):
            return ech[p].at[0] if goff is None else idxg[p]

        def _fire_gather(p):
            if goff is not None:
                for i in range(_K // 16):
                    idxg[p][pl.ds(i * 16, 16)] = ech[p][0, pl.ds(i * 16, 16)] + goff
            pltpu.async_copy(table.at[_gidx(p)], rows[p], sem_g[p])

        def _scale_scatter(p):
            def scale(g, c2):
                avec = plsc.bitcast(ech[p][2, pl.ds(g * 16, 16)], jnp.float32)
                e0 = g * 16
                for ei in range(16):
                    a = avec[ei]
                    rows[p][e0 + ei, pl.ds(0, 16)] = rows[p][e0 + ei, pl.ds(0, 16)] * a
                    rows[p][e0 + ei, pl.ds(16, 16)] = rows[p][e0 + ei, pl.ds(16, 16)] * a
                return c2

            lax.fori_loop(0, _K // 16, scale, 0)
            pltpu.sync_copy(rows[p], acc.at[ech[p].at[1]], add=True)

        def body(j, p):
            @pl.when(j + 1 < nchunks)
            def _():
                pltpu.make_async_copy(ed_hbm.at[sid, 0], ech[1 - p],
                                      sem_e[1 - p]).wait()
                _fire_gather(1 - p)

            pltpu.make_async_copy(table.at[_gidx(p)], rows[p], sem_g[p]).wait()
            _scale_scatter(p)

            @pl.when(j + 2 < nchunks)
            def _():
                pltpu.async_copy(ed_hbm.at[sid, j + 2], ech[p], sem_e[p])

        pltpu.sync_copy(ed_hbm.at[sid, 0], ech[0])
        _fire_gather(0)
        pltpu.async_copy(ed_hbm.at[sid, 1], ech[1], sem_e[1])

        def pair(j2, c):
            body(j2 * 2, 0)
            body(j2 * 2 + 1, 1)
            return c

        lax.fori_loop(0, nchunks // 2, pair, 0)

    _zero_hid()
    _zero_out()
    _drain_zero(hid_blocks, hid_acc)
    plsc.subcore_barrier()

    for mi in range(ncombo):
        m = core * ncombo + mi

        _edge_pass(x_hbm, edd_hbm, hid_acc, m * _ND)
        plsc.subcore_barrier()

        _drain_zero(out_blocks, out_acc)
        plsc.subcore_barrier()

        _edge_pass(hid_acc, edu_hbm, out_acc, None)
        plsc.subcore_barrier()

        if mi + 1 < ncombo:
            _zero_hid()

        _burst(out_blocks, lambda off: pltpu.async_copy(
            out_acc.at[pl.ds(off, _ZB)],
            out_hbm.at[pl.ds(m * _ND + off, _ZB)], sem_f))
        _drain(out_blocks, lambda: pltpu.make_async_copy(
            out_acc.at[pl.ds(0, _ZB)],
            out_hbm.at[pl.ds(0, _ZB)], sem_f))
        plsc.subcore_barrier()

        if mi + 1 < ncombo:
            _zero_out()
            _drain_zero(hid_blocks, hid_acc)
            plsc.subcore_barrier()


def kernel(x, edge_attr_down, edge_attr_up, edge_index_down, edge_index_up):
    batch, _, ens, nd, f = x.shape
    be = batch * ens
    ncombo = _NC * be // _NCORE
    e = edge_index_down.shape[1]
    nchunks = -(-e // (_NSUB * _K))
    nchunks += nchunks % 2
    epad = _NSUB * _K * nchunks

    def prep(edge_index, edge_attr):
        src = jnp.pad(edge_index[0].astype(jnp.int32), (0, epad - e))
        dst = jnp.pad(edge_index[1].astype(jnp.int32), (0, epad - e))
        attr = jax.lax.bitcast_convert_type(
            jnp.pad(edge_attr, (0, epad - e)), jnp.int32)
        ed = jnp.stack([src, dst, attr], axis=1)
        ed = ed.reshape(_NSUB, nchunks, _K, 3)
        return ed.transpose(0, 1, 3, 2)

    edd = prep(edge_index_down, edge_attr_down)
    edu = prep(edge_index_up, edge_attr_up)

    flat = x[:, -1].reshape(be, nd, _NC, _FC)
    x_t = flat.transpose(2, 0, 1, 3).reshape(_NC * be * nd, _FC)

    mesh = plsc.VectorSubcoreMesh(core_axis_name="c", subcore_axis_name="s")
    body = functools.partial(_sc_body, nchunks, ncombo)
    out = pl.kernel(
        body,
        out_type=jax.ShapeDtypeStruct((_NC * be * nd, _FC), jnp.float32),
        mesh=mesh,
        compiler_params=pltpu.CompilerParams(use_tc_tiling_on_sc=False,
                                             needs_layout_passes=False),
        scratch_types=[
            pltpu.VMEM_SHARED((_NH, _FC), jnp.float32),
            pltpu.VMEM_SHARED((_ND, _FC), jnp.float32),
            [pltpu.VMEM((3, _K), jnp.int32), pltpu.VMEM((3, _K), jnp.int32)],
            [pltpu.VMEM((_K,), jnp.int32), pltpu.VMEM((_K,), jnp.int32)],
            [pltpu.VMEM((_K, _FC), jnp.float32), pltpu.VMEM((_K, _FC), jnp.float32)],
            pltpu.VMEM((_ZB, _FC), jnp.float32),
            pltpu.SemaphoreType.DMA,
            pltpu.SemaphoreType.DMA,
            pltpu.SemaphoreType.DMA,
            pltpu.SemaphoreType.DMA,
            pltpu.SemaphoreType.DMA,
            pltpu.SemaphoreType.DMA,
        ],
    )(x_t, edd, edu)

    out = out.reshape(_NC, be, nd, _FC).transpose(1, 2, 0, 3)
    return out.reshape(batch, ens, nd, _F)


# speedup vs baseline: 24.9983x; 1.3849x over previous
import functools

import jax
import jax.numpy as jnp
from jax import lax
from jax.experimental import pallas as pl
from jax.experimental.pallas import tpu as pltpu, tpu_sc as plsc

_ND = 50000
_NH = 10000
_F = 256
_FC = 32
_NC = _F // _FC
_K = 128
_NSUB = 16
_NCORE = 2
_ZB = 40
_FB = 80


def _sc_body(nchunks, ncombo, be, t, x_hbm, edd_hbm, edu_hbm, out_hbm,
             hid_acc, out_acc, ech, idxg, rows, idxf, zeros_v,
             sem_e0, sem_e1, sem_g0, sem_g1, sem_z, sem_f):
    core = lax.axis_index("c")
    sid = lax.axis_index("s")
    sem_e = (sem_e0, sem_e1)
    sem_g = (sem_g0, sem_g1)

    zvec = jnp.zeros((16,), jnp.float32)

    def _zero_fill(i, c):
        zeros_v[i, pl.ds(0, 16)] = zvec
        zeros_v[i, pl.ds(16, 16)] = zvec
        return c

    lax.fori_loop(0, _ZB, _zero_fill, 0)

    hid_blocks = _NH // _ZB
    out_blocks = _ND // _ZB

    def _burst(nblocks, fire):
        def step(z, c):
            b = sid + z * _NSUB

            @pl.when(b < nblocks)
            def _():
                fire(pl.multiple_of(b * _ZB, 8))

            return c

        lax.fori_loop(0, -(-nblocks // _NSUB), step, 0)

    def _drain(nblocks, mk):
        cnt = (nblocks - sid + _NSUB - 1) // _NSUB

        def step(z, c):
            mk().wait()
            return c

        lax.fori_loop(0, cnt, step, 0)

    def _zero_hid():
        _burst(hid_blocks, lambda off: pltpu.async_copy(
            zeros_v, hid_acc.at[pl.ds(off, _ZB)], sem_z))

    def _zero_out():
        _burst(out_blocks, lambda off: pltpu.async_copy(
            zeros_v, out_acc.at[pl.ds(off, _ZB)], sem_z))

    def _drain_zero(nblocks, acc):
        _drain(nblocks, lambda: pltpu.make_async_copy(
            zeros_v, acc.at[pl.ds(0, _ZB)], sem_z))

    def _edge_pass(table, ed_hbm, acc, goff):
        def _gidx(p):
            return ech[p].at[0] if goff is None else idxg[p]

        def _fire_gather(p):
            if goff is not None:
                for i in range(_K // 16):
                    idxg[p][pl.ds(i * 16, 16)] = (
                        ech[p][0, pl.ds(i * 16, 16)] * _NC + goff)
            pltpu.async_copy(table.at[_gidx(p)], rows[p], sem_g[p])

        def _scale_scatter(p):
            def scale(g, c2):
                avec = plsc.bitcast(ech[p][2, pl.ds(g * 16, 16)], jnp.float32)
                e0 = g * 16
                for ei in range(16):
                    a = avec[ei]
                    rows[p][e0 + ei, pl.ds(0, 16)] = rows[p][e0 + ei, pl.ds(0, 16)] * a
                    rows[p][e0 + ei, pl.ds(16, 16)] = rows[p][e0 + ei, pl.ds(16, 16)] * a
                return c2

            lax.fori_loop(0, _K // 16, scale, 0)
            pltpu.sync_copy(rows[p], acc.at[ech[p].at[1]], add=True)

        def body(j, p):
            @pl.when(j + 1 < nchunks)
            def _():
                pltpu.make_async_copy(ed_hbm.at[sid, 0], ech[1 - p],
                                      sem_e[1 - p]).wait()
                _fire_gather(1 - p)

            pltpu.make_async_copy(table.at[_gidx(p)], rows[p], sem_g[p]).wait()
            _scale_scatter(p)

            @pl.when(j + 2 < nchunks)
            def _():
                pltpu.async_copy(ed_hbm.at[sid, j + 2], ech[p], sem_e[p])

        pltpu.sync_copy(ed_hbm.at[sid, 0], ech[0])
        _fire_gather(0)
        pltpu.async_copy(ed_hbm.at[sid, 1], ech[1], sem_e[1])

        def pair(j2, c):
            body(j2 * 2, 0)
            body(j2 * 2 + 1, 1)
            return c

        lax.fori_loop(0, nchunks // 2, pair, 0)

    _zero_hid()
    _zero_out()
    _drain_zero(hid_blocks, hid_acc)
    plsc.subcore_barrier()

    iota16 = lax.iota(jnp.int32, 16)

    def _flush(m):
        c_idx = m // be
        b_idx = m - c_idx * be
        foff = b_idx * _ND * _NC + c_idx
        nblocks = _ND // _FB

        def fire(z, p):
            b = sid + z * _NSUB

            @pl.when(b < nblocks)
            def _():
                node0 = b * _FB
                pltpu.sync_copy(out_acc.at[pl.ds(pl.multiple_of(node0, 8), _FB)],
                                rows[p].at[pl.ds(0, _FB)])
                for i in range(_FB // 16):
                    idxf[p][pl.ds(i * 16, 16)] = (
                        (iota16 + (node0 + i * 16)) * _NC + foff)
                pltpu.async_copy(rows[p].at[pl.ds(0, _FB)],
                                 out_hbm.at[idxf[p]], sem_f)

        def wait(z, p):
            b = sid + z * _NSUB

            @pl.when(b < nblocks)
            def _():
                pltpu.make_async_copy(rows[p].at[pl.ds(0, _FB)],
                                      out_hbm.at[idxf[p]], sem_f).wait()

        nz = -(-nblocks // _NSUB)
        fire(0, 0)

        def step(z2, c):
            z = z2 * 2
            fire(z + 1, 1)
            wait(z, 0)
            fire(z + 2, 0)
            wait(z + 1, 1)
            return c

        lax.fori_loop(0, nz // 2, step, 0)

    for mi in range(ncombo):
        m = core * ncombo + mi

        c_idx = m // be
        b_idx = m - c_idx * be
        goff = (b_idx * t + (t - 1)) * _ND * _NC + c_idx
        _edge_pass(x_hbm, edd_hbm, hid_acc, goff)
        plsc.subcore_barrier()

        _drain_zero(out_blocks, out_acc)
        plsc.subcore_barrier()

        _edge_pass(hid_acc, edu_hbm, out_acc, None)
        plsc.subcore_barrier()

        if mi + 1 < ncombo:
            _zero_hid()

        _flush(m)
        plsc.subcore_barrier()

        if mi + 1 < ncombo:
            _zero_out()
            _drain_zero(hid_blocks, hid_acc)
            plsc.subcore_barrier()


def kernel(x, edge_attr_down, edge_attr_up, edge_index_down, edge_index_up):
    batch, _, ens, nd, f = x.shape
    be = batch * ens
    ncombo = _NC * be // _NCORE
    e = edge_index_down.shape[1]
    nchunks = -(-e // (_NSUB * _K))
    nchunks += nchunks % 2
    epad = _NSUB * _K * nchunks

    def prep(edge_index, edge_attr):
        src = jnp.pad(edge_index[0].astype(jnp.int32), (0, epad - e))
        dst = jnp.pad(edge_index[1].astype(jnp.int32), (0, epad - e))
        attr = jax.lax.bitcast_convert_type(
            jnp.pad(edge_attr, (0, epad - e)), jnp.int32)
        ed = jnp.stack([src, dst, attr], axis=1)
        ed = ed.reshape(_NSUB, nchunks, _K, 3)
        return ed.transpose(0, 1, 3, 2)

    edd = prep(edge_index_down, edge_attr_down)
    edu = prep(edge_index_up, edge_attr_up)

    x_rows = x.reshape(-1, _FC)
    t = x.shape[1]

    mesh = plsc.VectorSubcoreMesh(core_axis_name="c", subcore_axis_name="s")
    body = functools.partial(_sc_body, nchunks, ncombo, be, t)
    out = pl.kernel(
        body,
        out_type=jax.ShapeDtypeStruct((be * nd * _NC, _FC), jnp.float32),
        mesh=mesh,
        compiler_params=pltpu.CompilerParams(use_tc_tiling_on_sc=False,
                                             needs_layout_passes=False),
        scratch_types=[
            pltpu.VMEM_SHARED((_NH, _FC), jnp.float32),
            pltpu.VMEM_SHARED((_ND, _FC), jnp.float32),
            [pltpu.VMEM((3, _K), jnp.int32), pltpu.VMEM((3, _K), jnp.int32)],
            [pltpu.VMEM((_K,), jnp.int32), pltpu.VMEM((_K,), jnp.int32)],
            [pltpu.VMEM((_K, _FC), jnp.float32), pltpu.VMEM((_K, _FC), jnp.float32)],
            [pltpu.VMEM((_FB,), jnp.int32), pltpu.VMEM((_FB,), jnp.int32)],
            pltpu.VMEM((_ZB, _FC), jnp.float32),
            pltpu.SemaphoreType.DMA,
            pltpu.SemaphoreType.DMA,
            pltpu.SemaphoreType.DMA,
            pltpu.SemaphoreType.DMA,
            pltpu.SemaphoreType.DMA,
            pltpu.SemaphoreType.DMA,
        ],
    )(x_rows, edd, edu)

    return out.reshape(batch, ens, nd, _F)


# speedup vs baseline: 27.0430x; 1.0818x over previous
import functools

import jax
import jax.numpy as jnp
from jax import lax
from jax.experimental import pallas as pl
from jax.experimental.pallas import tpu as pltpu, tpu_sc as plsc

_ND = 50000
_NH = 10000
_F = 256
_FC = 32
_NC = _F // _FC
_K = 128
_NSUB = 16
_NCORE = 2
_ZB = 1000
_FB = 80
_ED = 4


def _sc_body(nchunks, ncombo, be, t, x_hbm, edd_hbm, edu_hbm, z_hbm, out_hbm,
             hid_acc, out_acc, eb, idxg, rows, dstb, idxf,
             seme, semg, sem_z, sem_f):
    core = lax.axis_index("c")
    sid = lax.axis_index("s")

    hid_blocks = _NH // _ZB
    out_blocks = _ND // _ZB

    def _zero_fire(nblocks, acc):
        def step(z, c):
            b = sid + z * _NSUB

            @pl.when(b < nblocks)
            def _():
                pltpu.async_copy(z_hbm.at[pl.ds(0, _ZB)],
                                 acc.at[pl.ds(pl.multiple_of(b * _ZB, 8), _ZB)],
                                 sem_z)

            return c

        lax.fori_loop(0, -(-nblocks // _NSUB), step, 0)

    def _zero_drain(nblocks, acc):
        cnt = (nblocks - sid + _NSUB - 1) // _NSUB

        def step(z, c):
            pltpu.make_async_copy(z_hbm.at[pl.ds(0, _ZB)],
                                  acc.at[pl.ds(0, _ZB)], sem_z).wait()
            return c

        lax.fori_loop(0, cnt, step, 0)

    def _edge_pass(table, ed_hbm, acc, mult, goff):
        def _build_gidx(g, q):
            for i in range(_K // 16):
                v = eb[q][0, pl.ds(i * 16, 16)] & 0xFFFF
                if mult != 1:
                    v = v * mult
                idxg[g][pl.ds(i * 16, 16)] = v + goff

        def _fire_gather(g):
            pltpu.async_copy(table.at[idxg[g]], rows[g], semg[g])

        def _scale_scatter(r, q):
            def scale(g2, c2):
                avec = plsc.bitcast(eb[q][1, pl.ds(g2 * 16, 16)], jnp.float32)
                e0 = g2 * 16
                for ei in range(16):
                    a = avec[ei]
                    rows[r][e0 + ei, pl.ds(0, 16)] = rows[r][e0 + ei, pl.ds(0, 16)] * a
                    rows[r][e0 + ei, pl.ds(16, 16)] = rows[r][e0 + ei, pl.ds(16, 16)] * a
                return c2

            lax.fori_loop(0, _K // 16, scale, 0)
            for i in range(_K // 16):
                dstb[pl.ds(i * 16, 16)] = (eb[q][0, pl.ds(i * 16, 16)] >> 16) & 0xFFFF
            pltpu.sync_copy(rows[r], acc.at[dstb], add=True)

        def body(j, b):
            p, q = b % 2, b
            qn, pn = (b + 1) % _ED, (b + 1) % 2

            @pl.when(j + 1 < nchunks)
            def _():
                pltpu.make_async_copy(ed_hbm.at[sid, 0], eb[qn], seme[qn]).wait()
                _build_gidx(pn, qn)
                _fire_gather(pn)

            pltpu.make_async_copy(table.at[idxg[p]], rows[p], semg[p]).wait()
            _scale_scatter(p, q)

            @pl.when(j + 3 < nchunks)
            def _():
                pltpu.async_copy(ed_hbm.at[sid, j + 3], eb[(b + 3) % _ED],
                                 seme[(b + 3) % _ED])

        pltpu.sync_copy(ed_hbm.at[sid, 0], eb[0])
        pltpu.async_copy(ed_hbm.at[sid, 1], eb[1], seme[1])
        pltpu.async_copy(ed_hbm.at[sid, 2], eb[2], seme[2])
        _build_gidx(0, 0)
        _fire_gather(0)

        def quad(j4, c):
            for b in range(_ED):
                body(j4 * _ED + b, b)
            return c

        lax.fori_loop(0, nchunks // _ED, quad, 0)

    iota16 = lax.iota(jnp.int32, 16)

    def _flush(m):
        c_idx = m // be
        b_idx = m - c_idx * be
        foff = b_idx * _ND * _NC + c_idx
        nblocks = _ND // _FB

        def fire(z, p):
            b = sid + z * _NSUB

            @pl.when(b < nblocks)
            def _():
                node0 = b * _FB
                pltpu.sync_copy(out_acc.at[pl.ds(pl.multiple_of(node0, 8), _FB)],
                                rows[p].at[pl.ds(0, _FB)])
                for i in range(_FB // 16):
                    idxf[p][pl.ds(i * 16, 16)] = (
                        (iota16 + (node0 + i * 16)) * _NC + foff)
                pltpu.async_copy(rows[p].at[pl.ds(0, _FB)],
                                 out_hbm.at[idxf[p]], sem_f)

        def wait(z, p):
            b = sid + z * _NSUB

            @pl.when(b < nblocks)
            def _():
                pltpu.make_async_copy(rows[p].at[pl.ds(0, _FB)],
                                      out_hbm.at[idxf[p]], sem_f).wait()

        nz = -(-nblocks // _NSUB)
        fire(0, 0)

        def step(z2, c):
            z = z2 * 2
            fire(z + 1, 1)
            wait(z, 0)
            fire(z + 2, 0)
            wait(z + 1, 1)
            return c

        lax.fori_loop(0, nz // 2, step, 0)

    _zero_fire(hid_blocks, hid_acc)
    _zero_fire(out_blocks, out_acc)
    _zero_drain(hid_blocks, hid_acc)
    plsc.subcore_barrier()

    def combo(mi, carry):
        m = core * ncombo + mi
        c_idx = m // be
        b_idx = m - c_idx * be

        goff = (b_idx * t + (t - 1)) * _ND * _NC + c_idx
        _edge_pass(x_hbm, edd_hbm, hid_acc, _NC, goff)
        plsc.subcore_barrier()

        _zero_drain(out_blocks, out_acc)
        plsc.subcore_barrier()

        _edge_pass(hid_acc, edu_hbm, out_acc, 1, 0)
        plsc.subcore_barrier()

        @pl.when(mi + 1 < ncombo)
        def _():
            _zero_fire(hid_blocks, hid_acc)

        _flush(m)
        plsc.subcore_barrier()

        @pl.when(mi + 1 < ncombo)
        def _():
            _zero_fire(out_blocks, out_acc)
            _zero_drain(hid_blocks, hid_acc)

        plsc.subcore_barrier()
        return carry

    lax.fori_loop(0, ncombo, combo, 0)


def kernel(x, edge_attr_down, edge_attr_up, edge_index_down, edge_index_up):
    batch, _, ens, nd, f = x.shape
    be = batch * ens
    ncombo = _NC * be // _NCORE
    e = edge_index_down.shape[1]
    nchunks = -(-e // (_NSUB * _K))
    nchunks += (-nchunks) % _ED
    epad = _NSUB * _K * nchunks

    def prep(edge_index, edge_attr):
        src = jnp.pad(edge_index[0].astype(jnp.int32), (0, epad - e))
        dst = jnp.pad(edge_index[1].astype(jnp.int32), (0, epad - e))
        packed = src | (dst << 16)
        attr = jax.lax.bitcast_convert_type(
            jnp.pad(edge_attr, (0, epad - e)), jnp.int32)
        ed = jnp.stack([packed.reshape(_NSUB, nchunks, _K),
                        attr.reshape(_NSUB, nchunks, _K)], axis=2)
        return ed

    edd = prep(edge_index_down, edge_attr_down)
    edu = prep(edge_index_up, edge_attr_up)

    x_rows = x.reshape(-1, _FC)
    t = x.shape[1]
    zeros_hbm = jnp.zeros((_ZB, _FC), jnp.float32)

    mesh = plsc.VectorSubcoreMesh(core_axis_name="c", subcore_axis_name="s")
    body = functools.partial(_sc_body, nchunks, ncombo, be, t)
    out = pl.kernel(
        body,
        out_type=jax.ShapeDtypeStruct((be * nd * _NC, _FC), jnp.float32),
        mesh=mesh,
        compiler_params=pltpu.CompilerParams(use_tc_tiling_on_sc=False,
                                             needs_layout_passes=False),
        scratch_types=[
            pltpu.VMEM_SHARED((_NH, _FC), jnp.float32),
            pltpu.VMEM_SHARED((_ND, _FC), jnp.float32),
            [pltpu.VMEM((2, _K), jnp.int32) for _ in range(_ED)],
            [pltpu.VMEM((_K,), jnp.int32) for _ in range(2)],
            [pltpu.VMEM((_K, _FC), jnp.float32) for _ in range(2)],
            pltpu.VMEM((_K,), jnp.int32),
            [pltpu.VMEM((_FB,), jnp.int32) for _ in range(2)],
            [pltpu.SemaphoreType.DMA for _ in range(_ED)],
            [pltpu.SemaphoreType.DMA for _ in range(2)],
            pltpu.SemaphoreType.DMA,
            pltpu.SemaphoreType.DMA,
        ],
    )(x_rows, edd, edu, zeros_hbm)

    return out.reshape(batch, ens, nd, _F)


# speedup vs baseline: 32.0615x; 1.1856x over previous
import functools

import jax
import jax.numpy as jnp
from jax import lax
from jax.experimental import pallas as pl
from jax.experimental.pallas import tpu as pltpu, tpu_sc as plsc

_ND = 50000
_NH = 10000
_F = 256
_FC = 32
_NC = _F // _FC
_K = 96
_NSUB = 16
_NCORE = 2
_ZB = 1000
_FB = 80


def _sc_body(nchunks, ncombo, be, t, x_hbm, pkd_hbm, atd_hbm, pku_hbm,
             atu_hbm, z_hbm, out_hbm,
             hid_acc, out_acc, pkb, atb, idxg, rows, dstb, idxf,
             seme, semg, sems, sem_z, sem_f):
    core = lax.axis_index("c")
    sid = lax.axis_index("s")

    hid_blocks = _NH // _ZB
    out_blocks = _ND // _ZB

    def _zero_fire(nblocks, acc):
        def step(z, c):
            b = sid + z * _NSUB

            @pl.when(b < nblocks)
            def _():
                pltpu.async_copy(z_hbm.at[pl.ds(0, _ZB)],
                                 acc.at[pl.ds(pl.multiple_of(b * _ZB, 8), _ZB)],
                                 sem_z)

            return c

        lax.fori_loop(0, -(-nblocks // _NSUB), step, 0)

    def _zero_drain(nblocks, acc):
        cnt = (nblocks - sid + _NSUB - 1) // _NSUB

        def step(z, c):
            pltpu.make_async_copy(z_hbm.at[pl.ds(0, _ZB)],
                                  acc.at[pl.ds(0, _ZB)], sem_z).wait()
            return c

        lax.fori_loop(0, cnt, step, 0)

    def _edge_pass(table, pk_hbm, at_hbm, acc, mult, goff):
        def _fire_pair(q):
            pltpu.async_copy(pk_hbm.at[sid, pl.ds(q * 2, 2)], pkb[q % 3],
                             seme[q % 3])
            pltpu.async_copy(at_hbm.at[sid, pl.ds(q * 2, 2)], atb[q % 3],
                             seme[q % 3])

        def _wait_pair(q):
            pltpu.make_async_copy(pk_hbm.at[sid, pl.ds(0, 2)], pkb[q % 3],
                                  seme[q % 3]).wait()
            pltpu.make_async_copy(at_hbm.at[sid, pl.ds(0, 2)], atb[q % 3],
                                  seme[q % 3]).wait()

        def _fire_gather(g, qs, h):
            for i in range(_K // 16):
                v = pkb[qs][h, pl.ds(i * 16, 16)] & 0xFFFF
                if mult != 1:
                    v = v * mult
                idxg[g][pl.ds(i * 16, 16)] = v + goff
            pltpu.async_copy(table.at[idxg[g]], rows[g], semg[g])

        def _wait_scatter(r):
            pltpu.make_async_copy(rows[r], acc.at[dstb[0]], sems[r]).wait()

        def _scale_scatter(r, qs, h):
            pltpu.make_async_copy(table.at[idxg[r]], rows[r], semg[r]).wait()

            def scale(g2, c2):
                avec = plsc.bitcast(atb[qs][h, pl.ds(g2 * 16, 16)], jnp.float32)
                e0 = g2 * 16
                for ei in range(16):
                    a = avec[ei]
                    rows[r][e0 + ei, pl.ds(0, 16)] = rows[r][e0 + ei, pl.ds(0, 16)] * a
                    rows[r][e0 + ei, pl.ds(16, 16)] = rows[r][e0 + ei, pl.ds(16, 16)] * a
                return c2

            lax.fori_loop(0, _K // 16, scale, 0)
            for i in range(_K // 16):
                dstb[h][pl.ds(i * 16, 16)] = (pkb[qs][h, pl.ds(i * 16, 16)] >> 16) & 0xFFFF
            pltpu.async_copy(rows[r], acc.at[dstb[h]], sems[r], add=True)

        def body(jg, b):
            j = jg + b
            h = b % 2
            q3 = (b // 2) % 3
            qn = (q3 + 1) % 3

            if h == 0:
                @pl.when(j + 2 < nchunks)
                def _():
                    _wait_pair((b // 2) + 1)

            if b == 0:
                @pl.when(jg > 0)
                def _():
                    _wait_scatter((b + 2) % 3)
            else:
                _wait_scatter((b + 2) % 3)

            @pl.when(j + 2 < nchunks)
            def _():
                _fire_gather((b + 2) % 3, qn, h)

            _scale_scatter(b % 3, q3, h)

            if h == 1:
                @pl.when(j + 5 < nchunks)
                def _():
                    _fire_pair_dyn(jg, (b // 2) + 3)

        def _fire_pair_dyn(jg, qrel):
            q_slot = qrel % 3
            pltpu.async_copy(pk_hbm.at[sid, pl.ds(jg + qrel * 2, 2)],
                             pkb[q_slot], seme[q_slot])
            pltpu.async_copy(at_hbm.at[sid, pl.ds(jg + qrel * 2, 2)],
                             atb[q_slot], seme[q_slot])

        pltpu.sync_copy(pk_hbm.at[sid, pl.ds(0, 2)], pkb[0])
        pltpu.sync_copy(at_hbm.at[sid, pl.ds(0, 2)], atb[0])
        _fire_pair(1)
        _fire_pair(2)
        _fire_gather(0, 0, 0)
        _fire_gather(1, 0, 1)

        def group(g6, c):
            jg = g6 * 6
            for b in range(6):
                body(jg, b)
            return c

        lax.fori_loop(0, nchunks // 6, group, 0)
        _wait_scatter((nchunks - 1) % 3)

    iota16 = lax.iota(jnp.int32, 16)

    def _flush(m):
        c_idx = m // be
        b_idx = m - c_idx * be
        foff = b_idx * _ND * _NC + c_idx
        nblocks = _ND // _FB

        def fire(z, p):
            b = sid + z * _NSUB

            @pl.when(b < nblocks)
            def _():
                node0 = b * _FB
                pltpu.sync_copy(out_acc.at[pl.ds(pl.multiple_of(node0, 8), _FB)],
                                rows[p].at[pl.ds(0, _FB)])
                for i in range(_FB // 16):
                    idxf[p][pl.ds(i * 16, 16)] = (
                        (iota16 + (node0 + i * 16)) * _NC + foff)
                pltpu.async_copy(rows[p].at[pl.ds(0, _FB)],
                                 out_hbm.at[idxf[p]], sem_f)

        def wait(z, p):
            b = sid + z * _NSUB

            @pl.when(b < nblocks)
            def _():
                pltpu.make_async_copy(rows[p].at[pl.ds(0, _FB)],
                                      out_hbm.at[idxf[p]], sem_f).wait()

        nz = -(-nblocks // _NSUB)
        fire(0, 0)

        def step(z2, c):
            z = z2 * 2
            fire(z + 1, 1)
            wait(z, 0)
            fire(z + 2, 0)
            wait(z + 1, 1)
            return c

        lax.fori_loop(0, nz // 2, step, 0)

    _zero_fire(hid_blocks, hid_acc)
    _zero_fire(out_blocks, out_acc)
    _zero_drain(hid_blocks, hid_acc)
    plsc.subcore_barrier()

    def combo(mi, carry):
        m = core * ncombo + mi
        c_idx = m // be
        b_idx = m - c_idx * be

        goff = (b_idx * t + (t - 1)) * _ND * _NC + c_idx
        _edge_pass(x_hbm, pkd_hbm, atd_hbm, hid_acc, _NC, goff)
        plsc.subcore_barrier()

        _zero_drain(out_blocks, out_acc)
        plsc.subcore_barrier()

        _edge_pass(hid_acc, pku_hbm, atu_hbm, out_acc, 1, 0)
        plsc.subcore_barrier()

        @pl.when(mi + 1 < ncombo)
        def _():
            _zero_fire(hid_blocks, hid_acc)

        _flush(m)
        plsc.subcore_barrier()

        @pl.when(mi + 1 < ncombo)
        def _():
            _zero_fire(out_blocks, out_acc)
            _zero_drain(hid_blocks, hid_acc)

        plsc.subcore_barrier()
        return carry

    lax.fori_loop(0, ncombo, combo, 0)


def kernel(x, edge_attr_down, edge_attr_up, edge_index_down, edge_index_up):
    batch, _, ens, nd, f = x.shape
    be = batch * ens
    ncombo = _NC * be // _NCORE
    e = edge_index_down.shape[1]
    nchunks = -(-e // (_NSUB * _K))
    nchunks += (-nchunks) % 6
    epad = _NSUB * _K * nchunks

    def prep(edge_index, edge_attr):
        src = jnp.pad(edge_index[0].astype(jnp.int32), (0, epad - e))
        dst = jnp.pad(edge_index[1].astype(jnp.int32), (0, epad - e))
        packed = (src | (dst << 16)).reshape(_NSUB, nchunks, _K)
        attr = jax.lax.bitcast_convert_type(
            jnp.pad(edge_attr, (0, epad - e)), jnp.int32).reshape(_NSUB, nchunks, _K)
        return packed, attr

    pkd, atd = prep(edge_index_down, edge_attr_down)
    pku, atu = prep(edge_index_up, edge_attr_up)

    x_rows = x.reshape(-1, _FC)
    t = x.shape[1]
    zeros_hbm = jnp.zeros((_ZB, _FC), jnp.float32)

    mesh = plsc.VectorSubcoreMesh(core_axis_name="c", subcore_axis_name="s")
    body = functools.partial(_sc_body, nchunks, ncombo, be, t)
    out = pl.kernel(
        body,
        out_type=jax.ShapeDtypeStruct((be * nd * _NC, _FC), jnp.float32),
        mesh=mesh,
        compiler_params=pltpu.CompilerParams(use_tc_tiling_on_sc=False,
                                             needs_layout_passes=False),
        scratch_types=[
            pltpu.VMEM_SHARED((_NH, _FC), jnp.float32),
            pltpu.VMEM_SHARED((_ND, _FC), jnp.float32),
            [pltpu.VMEM((2, _K), jnp.int32) for _ in range(3)],
            [pltpu.VMEM((2, _K), jnp.int32) for _ in range(3)],
            [pltpu.VMEM((_K,), jnp.int32) for _ in range(3)],
            [pltpu.VMEM((_K, _FC), jnp.float32) for _ in range(3)],
            [pltpu.VMEM((_K,), jnp.int32) for _ in range(2)],
            [pltpu.VMEM((_FB,), jnp.int32) for _ in range(2)],
            [pltpu.SemaphoreType.DMA for _ in range(3)],
            [pltpu.SemaphoreType.DMA for _ in range(3)],
            [pltpu.SemaphoreType.DMA for _ in range(3)],
            pltpu.SemaphoreType.DMA,
            pltpu.SemaphoreType.DMA,
        ],
    )(x_rows, pkd, atd, pku, atu, zeros_hbm)

    return out.reshape(batch, ens, nd, _F)


# speedup vs baseline: 32.1726x; 1.0035x over previous
import functools

import jax
import jax.numpy as jnp
from jax import lax
from jax.experimental import pallas as pl
from jax.experimental.pallas import tpu as pltpu, tpu_sc as plsc

_ND = 50000
_NH = 10000
_F = 256
_FC = 32
_NC = _F // _FC
_K = 96
_NSUB = 16
_NCORE = 2
_ZB = 1000
_FB = 80


def _sc_body(nchunks, ncombo, be, t, x_hbm, pkd_hbm, atd_hbm, pku_hbm,
             atu_hbm, z_hbm, out_hbm,
             hid_acc, out_acc, pkb, atb, idxg, rows, dstb, idxf,
             seme, semg, sems, sem_z, sem_f):
    core = lax.axis_index("c")
    sid = lax.axis_index("s")

    hid_blocks = _NH // _ZB
    out_blocks = _ND // _ZB

    def _zero_fire(nblocks, acc):
        def step(z, c):
            b = sid + z * _NSUB

            @pl.when(b < nblocks)
            def _():
                pltpu.async_copy(z_hbm.at[pl.ds(0, _ZB)],
                                 acc.at[pl.ds(pl.multiple_of(b * _ZB, 8), _ZB)],
                                 sem_z)

            return c

        lax.fori_loop(0, -(-nblocks // _NSUB), step, 0)

    def _zero_drain(nblocks, acc):
        cnt = (nblocks - sid + _NSUB - 1) // _NSUB

        def step(z, c):
            pltpu.make_async_copy(z_hbm.at[pl.ds(0, _ZB)],
                                  acc.at[pl.ds(0, _ZB)], sem_z).wait()
            return c

        lax.fori_loop(0, cnt, step, 0)

    def _edge_pass(table, pk_hbm, at_hbm, acc, mult, goff):
        def _fire_pair(q):
            pltpu.async_copy(pk_hbm.at[sid, pl.ds(q * 2, 2)], pkb[q % 3],
                             seme[q % 3])
            pltpu.async_copy(at_hbm.at[sid, pl.ds(q * 2, 2)], atb[q % 3],
                             seme[q % 3])

        def _wait_pair(q):
            pltpu.make_async_copy(pk_hbm.at[sid, pl.ds(0, 2)], pkb[q % 3],
                                  seme[q % 3]).wait()
            pltpu.make_async_copy(at_hbm.at[sid, pl.ds(0, 2)], atb[q % 3],
                                  seme[q % 3]).wait()

        def _fire_gather(g, qs, h):
            for i in range(_K // 16):
                v = pkb[qs][h, pl.ds(i * 16, 16)] & 0xFFFF
                if mult != 1:
                    v = v * mult
                idxg[g][pl.ds(i * 16, 16)] = v + goff
            pltpu.async_copy(table.at[idxg[g]], rows[g], semg[g])

        def _wait_scatter(r):
            pltpu.make_async_copy(rows[r], acc.at[dstb[0]], sems[r]).wait()

        def _scale_scatter(r, qs, h):
            pltpu.make_async_copy(table.at[idxg[r]], rows[r], semg[r]).wait()

            def scale(g2, c2):
                avec = plsc.bitcast(atb[qs][h, pl.ds(g2 * 16, 16)], jnp.float32)
                e0 = g2 * 16
                for ei in range(16):
                    a = avec[ei]
                    rows[r][e0 + ei, pl.ds(0, 16)] = rows[r][e0 + ei, pl.ds(0, 16)] * a
                    rows[r][e0 + ei, pl.ds(16, 16)] = rows[r][e0 + ei, pl.ds(16, 16)] * a
                return c2

            lax.fori_loop(0, _K // 16, scale, 0)
            for i in range(_K // 16):
                dstb[h][pl.ds(i * 16, 16)] = (pkb[qs][h, pl.ds(i * 16, 16)] >> 16) & 0xFFFF
            pltpu.async_copy(rows[r], acc.at[dstb[h]], sems[r], add=True)

        def body(jg, b):
            j = jg + b
            h = b % 2
            q3 = (b // 2) % 3
            qn = (q3 + 1) % 3

            if h == 0:
                @pl.when(j + 2 < nchunks)
                def _():
                    _wait_pair((b // 2) + 1)

            if b == 0:
                @pl.when(jg > 0)
                def _():
                    _wait_scatter((b + 2) % 3)
            else:
                _wait_scatter((b + 2) % 3)

            @pl.when(j + 2 < nchunks)
            def _():
                _fire_gather((b + 2) % 3, qn, h)

            _scale_scatter(b % 3, q3, h)

            if h == 1:
                @pl.when(j + 5 < nchunks)
                def _():
                    _fire_pair_dyn(jg, (b // 2) + 3)

        def _fire_pair_dyn(jg, qrel):
            q_slot = qrel % 3
            pltpu.async_copy(pk_hbm.at[sid, pl.ds(jg + qrel * 2, 2)],
                             pkb[q_slot], seme[q_slot])
            pltpu.async_copy(at_hbm.at[sid, pl.ds(jg + qrel * 2, 2)],
                             atb[q_slot], seme[q_slot])

        pltpu.sync_copy(pk_hbm.at[sid, pl.ds(0, 2)], pkb[0])
        pltpu.sync_copy(at_hbm.at[sid, pl.ds(0, 2)], atb[0])
        _fire_pair(1)
        _fire_pair(2)
        _fire_gather(0, 0, 0)
        _fire_gather(1, 0, 1)

        def group(g6, c):
            jg = g6 * 6
            for b in range(6):
                body(jg, b)
            return c

        lax.fori_loop(0, nchunks // 6, group, 0)
        _wait_scatter((nchunks - 1) % 3)

    iota16 = lax.iota(jnp.int32, 16)

    def _flush(m):
        c_idx = m // be
        b_idx = m - c_idx * be
        foff = b_idx * _ND * _NC + c_idx
        nblocks = _ND // _FB

        def fire_copyin(z, r):
            b = sid + z * _NSUB

            @pl.when(b < nblocks)
            def _():
                pltpu.async_copy(
                    out_acc.at[pl.ds(pl.multiple_of(b * _FB, 8), _FB)],
                    rows[r].at[pl.ds(0, _FB)], semg[r])

        def body(z, r, h):
            b = sid + z * _NSUB

            @pl.when(b < nblocks)
            def _():
                pltpu.make_async_copy(out_acc.at[pl.ds(0, _FB)],
                                      rows[r].at[pl.ds(0, _FB)], semg[r]).wait()
                node0 = b * _FB
                for i in range(_FB // 16):
                    idxf[h][pl.ds(i * 16, 16)] = (
                        (iota16 + (node0 + i * 16)) * _NC + foff)
                pltpu.async_copy(rows[r].at[pl.ds(0, _FB)],
                                 out_hbm.at[idxf[h]], sems[r])

            bp = sid + (z - 1) * _NSUB

            @pl.when((z >= 1) & (bp < nblocks))
            def _():
                pltpu.make_async_copy(rows[(r + 2) % 3].at[pl.ds(0, _FB)],
                                      out_hbm.at[idxf[1 - h]],
                                      sems[(r + 2) % 3]).wait()

            fire_copyin(z + 2, (r + 2) % 3)

        fire_copyin(0, 0)
        fire_copyin(1, 1)
        nz6 = (-(-nblocks // _NSUB) + 2 + 5) // 6

        def step(z6, c):
            for u in range(6):
                body_z = z6 * 6 + u
                body(body_z, u % 3, u % 2)
            return c

        lax.fori_loop(0, nz6, step, 0)

    _zero_fire(hid_blocks, hid_acc)
    _zero_fire(out_blocks, out_acc)
    _zero_drain(hid_blocks, hid_acc)
    plsc.subcore_barrier()

    def combo(mi, carry):
        m = core * ncombo + mi
        c_idx = m // be
        b_idx = m - c_idx * be

        goff = (b_idx * t + (t - 1)) * _ND * _NC + c_idx
        _edge_pass(x_hbm, pkd_hbm, atd_hbm, hid_acc, _NC, goff)
        plsc.subcore_barrier()

        _zero_drain(out_blocks, out_acc)
        plsc.subcore_barrier()

        _edge_pass(hid_acc, pku_hbm, atu_hbm, out_acc, 1, 0)
        plsc.subcore_barrier()

        @pl.when(mi + 1 < ncombo)
        def _():
            _zero_fire(hid_blocks, hid_acc)

        _flush(m)
        plsc.subcore_barrier()

        @pl.when(mi + 1 < ncombo)
        def _():
            _zero_fire(out_blocks, out_acc)
            _zero_drain(hid_blocks, hid_acc)

        plsc.subcore_barrier()
        return carry

    lax.fori_loop(0, ncombo, combo, 0)


def kernel(x, edge_attr_down, edge_attr_up, edge_index_down, edge_index_up):
    batch, _, ens, nd, f = x.shape
    be = batch * ens
    ncombo = _NC * be // _NCORE
    e = edge_index_down.shape[1]
    nchunks = -(-e // (_NSUB * _K))
    nchunks += (-nchunks) % 6
    epad = _NSUB * _K * nchunks

    def prep(edge_index, edge_attr):
        src = jnp.pad(edge_index[0].astype(jnp.int32), (0, epad - e))
        dst = jnp.pad(edge_index[1].astype(jnp.int32), (0, epad - e))
        packed = (src | (dst << 16)).reshape(_NSUB, nchunks, _K)
        attr = jax.lax.bitcast_convert_type(
            jnp.pad(edge_attr, (0, epad - e)), jnp.int32).reshape(_NSUB, nchunks, _K)
        return packed, attr

    pkd, atd = prep(edge_index_down, edge_attr_down)
    pku, atu = prep(edge_index_up, edge_attr_up)

    x_rows = x.reshape(-1, _FC)
    t = x.shape[1]
    zeros_hbm = jnp.zeros((_ZB, _FC), jnp.float32)

    mesh = plsc.VectorSubcoreMesh(core_axis_name="c", subcore_axis_name="s")
    body = functools.partial(_sc_body, nchunks, ncombo, be, t)
    out = pl.kernel(
        body,
        out_type=jax.ShapeDtypeStruct((be * nd * _NC, _FC), jnp.float32),
        mesh=mesh,
        compiler_params=pltpu.CompilerParams(use_tc_tiling_on_sc=False,
                                             needs_layout_passes=False),
        scratch_types=[
            pltpu.VMEM_SHARED((_NH, _FC), jnp.float32),
            pltpu.VMEM_SHARED((_ND, _FC), jnp.float32),
            [pltpu.VMEM((2, _K), jnp.int32) for _ in range(3)],
            [pltpu.VMEM((2, _K), jnp.int32) for _ in range(3)],
            [pltpu.VMEM((_K,), jnp.int32) for _ in range(3)],
            [pltpu.VMEM((_K, _FC), jnp.float32) for _ in range(3)],
            [pltpu.VMEM((_K,), jnp.int32) for _ in range(2)],
            [pltpu.VMEM((_FB,), jnp.int32) for _ in range(2)],
            [pltpu.SemaphoreType.DMA for _ in range(3)],
            [pltpu.SemaphoreType.DMA for _ in range(3)],
            [pltpu.SemaphoreType.DMA for _ in range(3)],
            pltpu.SemaphoreType.DMA,
            pltpu.SemaphoreType.DMA,
        ],
    )(x_rows, pkd, atd, pku, atu, zeros_hbm)

    return out.reshape(batch, ens, nd, _F)
